# Initial kernel scaffold; baseline (speedup 1.0000x reference)
import jax
import jax.numpy as jnp
from jax.experimental import pallas as pl


def kernel(x, state, ln_gamma, ln_beta, Wq, bq):
    raise NotImplementedError("write your pallas kernel here")


# speedup vs baseline: 1.2724x; 1.2724x over previous
import functools

import jax
import jax.numpy as jnp
from jax.experimental import pallas as pl

_B = 256
_D = 256
_S = 1024
_LN_EPS = 1e-5
_SCALE = 1.0 * (_D ** -0.5)


def _q_kernel(x_ref, g_ref, b_ref, wq_ref, bq_ref, q_ref):
    x = x_ref[...]
    mu = jnp.mean(x, axis=-1, keepdims=True)
    var = jnp.mean((x - mu) ** 2, axis=-1, keepdims=True)
    xn = (x - mu) * jax.lax.rsqrt(var + _LN_EPS) * g_ref[...] + b_ref[...]
    q_ref[...] = jax.lax.dot_general(
        xn, wq_ref[...], (((1,), (1,)), ((), ())),
        preferred_element_type=jnp.float32) + bq_ref[...]


def _read_kernel(q_ref, s_ref, o_ref, *, rows):
    q = q_ref[...]
    for r in range(rows):
        s = s_ref[r]
        qr = q[r:r + 1]
        scores = jax.lax.dot_general(
            qr, s, (((1,), (1,)), ((), ())),
            preferred_element_type=jnp.float32) * _SCALE
        m = jnp.max(scores, axis=-1, keepdims=True)
        e = jnp.exp(scores - m)
        attn = e / jnp.sum(e, axis=-1, keepdims=True)
        o_ref[r:r + 1] = jnp.dot(
            attn, s, preferred_element_type=jnp.float32)


@jax.jit
def kernel(x, state, ln_gamma, ln_beta, Wq, bq):
    g2 = ln_gamma.reshape(1, _D)
    b2 = ln_beta.reshape(1, _D)
    bq2 = bq.reshape(1, _D)

    q = pl.pallas_call(
        _q_kernel,
        out_shape=jax.ShapeDtypeStruct((_B, _D), jnp.float32),
    )(x, g2, b2, Wq, bq2)

    rows = 8
    out = pl.pallas_call(
        functools.partial(_read_kernel, rows=rows),
        grid=(_B // rows,),
        in_specs=[
            pl.BlockSpec((rows, _D), lambda i: (i, 0)),
            pl.BlockSpec((rows, _S, _D), lambda i: (i, 0, 0)),
        ],
        out_specs=pl.BlockSpec((rows, _D), lambda i: (i, 0)),
        out_shape=jax.ShapeDtypeStruct((_B, _D), jnp.float32),
    )(q, state)
    return out
